# baseline (speedup 1.0000x reference)
import dataclasses
import functools

import jax
import jax.numpy as jnp
from jax import lax
from jax.experimental import pallas as pl
from jax.experimental.pallas import tpu as pltpu
from jax.experimental.pallas import tpu_sc as plsc

N = 10000
E = 320000
H = 128
T = 4

NC = 2
NS = 16
NW = NC * NS

BATCH = 128
NB = 80
IDX_CHUNK = 8
RB = 64
RNB = 160
RIDX = 8
NBUF = 3
PAIRS_PAD = NW * NB * BATCH
ROWS_PER_TILE = 632
ACC_ROWS = NS * ROWS_PER_TILE
CNT_W = 16
DUMP = N
CROWS = NS * NB
CRPT = CROWS // NS

_MESH = plsc.VectorSubcoreMesh(core_axis_name="c", subcore_axis_name="s")
_CP = pltpu.CompilerParams()
if "needs_layout_passes" in pltpu.CompilerParams.__dataclass_fields__:
    _CP = dataclasses.replace(_CP, needs_layout_passes=False)


def _yy_body(emb_ref, w_ref, out_ref):
    out_ref[...] = lax.dot_general(
        emb_ref[...], w_ref[0],
        (((1,), (1,)), ((), ())),
        preferred_element_type=jnp.float32,
    )


def _yy_matmul(emb, wstk):
    blk = 400
    return pl.pallas_call(
        _yy_body,
        grid=(8, N // blk),
        in_specs=[
            pl.BlockSpec((blk, H), lambda k, n: (n, 0)),
            pl.BlockSpec((1, H, H), lambda k, n: (k, 0, 0)),
        ],
        out_specs=pl.BlockSpec((blk, H), lambda k, n: (k * (N // blk) + n, 0)),
        out_shape=jax.ShapeDtypeStruct((8 * N, H), jnp.float32),
    )(emb, wstk)


def _combine_body(p_ref, c_ref, yd_ref, o_ref):
    cnt = c_ref[0] + c_ref[1]
    acc = p_ref[0] + p_ref[1]
    for t in range(T):
        acc = acc + cnt[:, t:t + 1] * yd_ref[t]
    o_ref[...] = acc


def _combine(partials, cparts, ydst):
    blk = 400
    return pl.pallas_call(
        _combine_body,
        grid=(N // blk,),
        in_specs=[
            pl.BlockSpec((2, blk, H), lambda i: (0, i, 0)),
            pl.BlockSpec((2, blk, CNT_W), lambda i: (0, i, 0)),
            pl.BlockSpec((T, blk, H), lambda i: (0, i, 0)),
        ],
        out_specs=pl.BlockSpec((blk, H), lambda i: (i, 0)),
        out_shape=jax.ShapeDtypeStruct((N, H), jnp.float32),
    )(partials, cparts, ydst)


def _sc_rows(yy, g3, d3, c3, s3):
    @functools.partial(
        pl.kernel,
        compiler_params=_CP,
        out_type=[
            jax.ShapeDtypeStruct((NC, ACC_ROWS, H), jnp.float32),
            jax.ShapeDtypeStruct((NC, CROWS, H), jnp.float32),
        ],
        mesh=_MESH,
        scratch_types=[
            pltpu.VMEM((RIDX, RB), jnp.int32),
            pltpu.VMEM((RIDX, RB), jnp.int32),
            pltpu.VMEM((RIDX, RB), jnp.int32),
            pltpu.VMEM((RIDX, RB), jnp.int32),
            pltpu.VMEM((RB, H), jnp.float32),
        ] + [pltpu.VMEM((RB, H), jnp.float32) for _ in range(NBUF)] + [
            pltpu.VMEM_SHARED((ACC_ROWS, H), jnp.float32),
            pltpu.VMEM_SHARED((CROWS, H), jnp.float32),
        ] + [pltpu.SemaphoreType.DMA for _ in range(2 * NBUF)],
    )
    def k(yy_hbm, g_hbm, d_hbm, c_hbm, s_hbm, out_hbm, cnt_hbm,
          g_v, d_v, c_v, s_v, oh_v, *rest):
        bufs = rest[:NBUF]
        acc = rest[NBUF]
        cac = rest[NBUF + 1]
        gsem = rest[NBUF + 2:NBUF + 2 + NBUF]
        ssem = rest[NBUF + 2 + NBUF:]
        cid = lax.axis_index("c")
        sid = lax.axis_index("s")
        wid = cid * NS + sid
        ones = jnp.ones((16,), jnp.float32)

        zero = jnp.zeros((16,), jnp.float32)

        @pl.loop(0, RB)
        def _(r):
            @pl.loop(0, H // 16)
            def _(c):
                bufs[0][r, pl.ds(c * 16, 16)] = zero

        @pl.loop(0, RB)
        def _(r):
            @pl.loop(0, H // 16)
            def _(c):
                oh_v[r, pl.ds(c * 16, 16)] = zero

        for b in range(ROWS_PER_TILE // RB):
            pltpu.sync_copy(
                bufs[0], acc.at[pl.ds(sid * ROWS_PER_TILE + b * RB, RB)]
            )
        rtail = ROWS_PER_TILE - (ROWS_PER_TILE // RB) * RB
        if rtail:
            pltpu.sync_copy(
                bufs[0].at[pl.ds(0, rtail)],
                acc.at[pl.ds(sid * ROWS_PER_TILE + ROWS_PER_TILE - rtail,
                             rtail)],
            )
        pltpu.sync_copy(bufs[0], cac.at[pl.ds(sid * CRPT, RB)])
        pltpu.sync_copy(bufs[0].at[pl.ds(0, CRPT - RB)],
                        cac.at[pl.ds(sid * CRPT + RB, CRPT - RB)])

        plsc.subcore_barrier()

        def gather(j, b):
            return pltpu.make_async_copy(yy_hbm.at[g_v.at[j]], bufs[b], gsem[b])

        def scat(j, b):
            return pltpu.make_async_copy(bufs[b], acc.at[d_v.at[j]], ssem[b])

        @pl.loop(0, RNB // RIDX)
        def _(c):
            pltpu.sync_copy(g_hbm.at[wid, pl.ds(c * RIDX, RIDX)], g_v)
            pltpu.sync_copy(d_hbm.at[wid, pl.ds(c * RIDX, RIDX)], d_v)
            pltpu.sync_copy(c_hbm.at[wid, pl.ds(c * RIDX, RIDX)], c_v)
            pltpu.sync_copy(s_hbm.at[wid, pl.ds(c * RIDX, RIDX)], s_v)

            for j in range(NBUF):
                gather(j, j).start()
            for j in range(RIDX):
                b = j % NBUF
                if j > 0:
                    pb = (j - 1) % NBUF
                    scat(j - 1, pb).wait()
                    if j + NBUF - 1 < RIDX:
                        gather(j + NBUF - 1, pb).start()
                gather(j, b).wait()
                scat(j, b).start(add=True)
                for l in range(RB // 16):
                    rows = lax.iota(jnp.int32, 16) + (l * 16)
                    colv = c_v.at[j][pl.ds(l * 16, 16)]
                    plsc.addupdate_scatter(oh_v, [rows, colv], ones)
                pltpu.sync_copy(oh_v, cac.at[s_v.at[j]], add=True)
                for l in range(RB // 16):
                    rows = lax.iota(jnp.int32, 16) + (l * 16)
                    colv = c_v.at[j][pl.ds(l * 16, 16)]
                    plsc.addupdate_scatter(oh_v, [rows, colv], -ones)
            scat(RIDX - 1, (RIDX - 1) % NBUF).wait()

        plsc.subcore_barrier()

        for b in range(ROWS_PER_TILE // RB):
            base = sid * ROWS_PER_TILE + b * RB
            pltpu.sync_copy(
                acc.at[pl.ds(base, RB)], out_hbm.at[cid, pl.ds(base, RB)]
            )
        if rtail:
            base = sid * ROWS_PER_TILE + ROWS_PER_TILE - rtail
            pltpu.sync_copy(
                acc.at[pl.ds(base, rtail)], out_hbm.at[cid, pl.ds(base, rtail)]
            )
        pltpu.sync_copy(cac.at[pl.ds(sid * CRPT, CRPT)],
                        cnt_hbm.at[cid, pl.ds(sid * CRPT, CRPT)])

    return k(yy, g3, d3, c3, s3)


@jax.jit
def kernel(state_embedding, typed_edges, W):
    et = typed_edges[0] - 1
    src = typed_edges[1]
    dst = typed_edges[2]

    def shard(x, padval, nb, batch):
        pad = NW * nb * batch - E
        x = jnp.concatenate([x, jnp.full((pad,), padval, jnp.int32)])
        return x.reshape(nb, NW, batch).swapaxes(0, 1)

    g3 = shard(et * N + src, 0, RNB, RB)
    d3 = shard(dst, DUMP, RNB, RB)
    c3 = shard(((dst & 7) << 4) + et, 0, RNB, RB)
    s3 = shard(dst >> 3, DUMP >> 3, RNB, RB)

    wstk = jnp.concatenate([W[:, :, :H], W[:, :, H:]], axis=0)

    yy = _yy_matmul(state_embedding, wstk)
    partials, cparts = _sc_rows(yy, g3, d3, c3, s3)
    ydst = yy.reshape(8, N, H)[T:]
    cnt = cparts.reshape(2, CROWS * 8, CNT_W)
    return _combine(partials, cnt, ydst)


# speedup vs baseline: 1.1790x; 1.1790x over previous
import dataclasses
import functools

import jax
import jax.numpy as jnp
from jax import lax
from jax.experimental import pallas as pl
from jax.experimental.pallas import tpu as pltpu
from jax.experimental.pallas import tpu_sc as plsc

N = 10000
E = 320000
H = 128
T = 4

NC = 2
NS = 16
NW = NC * NS

BATCH = 128
NB = 80
IDX_CHUNK = 8
RB = 64
RNB = 160
RIDX = 16
NBUF = 4
ROWS_PER_TILE = 640
ACC_ROWS = NS * ROWS_PER_TILE
CNT_W = 16
CROWS = NS * NB
CRPT = CROWS // NS
DUMP = N

_MESH = plsc.VectorSubcoreMesh(core_axis_name="c", subcore_axis_name="s")
_CP = pltpu.CompilerParams()
if "needs_layout_passes" in pltpu.CompilerParams.__dataclass_fields__:
    _CP = dataclasses.replace(_CP, needs_layout_passes=False)


def _yy_body(emb_ref, w_ref, out_ref):
    out_ref[...] = lax.dot_general(
        emb_ref[...], w_ref[0],
        (((1,), (1,)), ((), ())),
        preferred_element_type=jnp.float32,
    )


def _yy_matmul(emb, wstk):
    blk = 400
    return pl.pallas_call(
        _yy_body,
        grid=(8, N // blk),
        in_specs=[
            pl.BlockSpec((blk, H), lambda k, n: (n, 0)),
            pl.BlockSpec((1, H, H), lambda k, n: (k, 0, 0)),
        ],
        out_specs=pl.BlockSpec((blk, H), lambda k, n: (k * (N // blk) + n, 0)),
        out_shape=jax.ShapeDtypeStruct((8 * N, H), jnp.float32),
    )(emb, wstk)


def _combine_body(p_ref, c_ref, yd_ref, o_ref):
    cnt = c_ref[0] + c_ref[1]
    acc = p_ref[0] + p_ref[1]
    for t in range(T):
        acc = acc + cnt[:, t:t + 1] * yd_ref[t]
    o_ref[...] = acc


def _combine(partials, cparts, ydst):
    blk = 400
    return pl.pallas_call(
        _combine_body,
        grid=(N // blk,),
        in_specs=[
            pl.BlockSpec((2, blk, H), lambda i: (0, i, 0)),
            pl.BlockSpec((2, blk, CNT_W), lambda i: (0, i, 0)),
            pl.BlockSpec((T, blk, H), lambda i: (0, i, 0)),
        ],
        out_specs=pl.BlockSpec((blk, H), lambda i: (i, 0)),
        out_shape=jax.ShapeDtypeStruct((N, H), jnp.float32),
    )(partials, cparts, ydst)


def _sc_rows(yy, g3, d3):
    @functools.partial(
        pl.kernel,
        compiler_params=_CP,
        out_type=jax.ShapeDtypeStruct((NC, ACC_ROWS, H), jnp.float32),
        mesh=_MESH,
        scratch_types=[
            pltpu.VMEM((RIDX, RB), jnp.int32),
            pltpu.VMEM((RIDX, RB), jnp.int32),
        ] + [pltpu.VMEM((RB, H), jnp.float32) for _ in range(NBUF)] + [
            pltpu.VMEM_SHARED((ACC_ROWS, H), jnp.float32),
        ] + [pltpu.SemaphoreType.DMA for _ in range(2 * NBUF)],
    )
    def k(yy_hbm, g_hbm, d_hbm, out_hbm, g_v, d_v, *rest):
        bufs = rest[:NBUF]
        acc = rest[NBUF]
        gsem = rest[NBUF + 1:NBUF + 1 + NBUF]
        ssem = rest[NBUF + 1 + NBUF:]
        cid = lax.axis_index("c")
        sid = lax.axis_index("s")
        wid = cid * NS + sid

        zero = jnp.zeros((16,), jnp.float32)

        @pl.loop(0, RB)
        def _(r):
            @pl.loop(0, H // 16)
            def _(c):
                bufs[0][r, pl.ds(c * 16, 16)] = zero

        @pl.loop(0, ROWS_PER_TILE // RB)
        def _(b):
            pltpu.sync_copy(
                bufs[0], acc.at[pl.ds(sid * ROWS_PER_TILE + b * RB, RB)]
            )

        plsc.subcore_barrier()

        def gather(j, b):
            return pltpu.make_async_copy(yy_hbm.at[g_v.at[j]], bufs[b], gsem[b])

        def scat(j, b):
            return pltpu.make_async_copy(bufs[b], acc.at[d_v.at[j]], ssem[b])

        @pl.loop(0, RNB // RIDX)
        def _(c):
            pltpu.sync_copy(g_hbm.at[wid, pl.ds(c * RIDX, RIDX)], g_v)
            pltpu.sync_copy(d_hbm.at[wid, pl.ds(c * RIDX, RIDX)], d_v)

            for j in range(NBUF):
                gather(j, j).start()
            for j in range(RIDX):
                b = j % NBUF
                if j > 0:
                    pb = (j - 1) % NBUF
                    scat(j - 1, pb).wait()
                    if j + NBUF - 1 < RIDX:
                        gather(j + NBUF - 1, pb).start()
                gather(j, b).wait()
                scat(j, b).start(add=True)
            scat(RIDX - 1, (RIDX - 1) % NBUF).wait()

        plsc.subcore_barrier()

        @pl.loop(0, ROWS_PER_TILE // BATCH)
        def _(b):
            base = sid * ROWS_PER_TILE + b * BATCH
            pltpu.sync_copy(
                acc.at[pl.ds(base, BATCH)], out_hbm.at[cid, pl.ds(base, BATCH)]
            )

    return k(yy, g3, d3)


def _sc_counts(c3, s3):
    @functools.partial(
        pl.kernel,
        compiler_params=_CP,
        out_type=jax.ShapeDtypeStruct((NC, CROWS, H), jnp.float32),
        mesh=_MESH,
        scratch_types=[
            pltpu.VMEM((IDX_CHUNK, BATCH), jnp.int32),
            pltpu.VMEM((IDX_CHUNK, BATCH), jnp.int32),
            pltpu.VMEM((BATCH, H), jnp.float32),
            pltpu.VMEM_SHARED((CROWS, H), jnp.float32),
        ],
    )
    def k(c_hbm, s_hbm, cnt_hbm, c_v, s_v, oh_v, cac):
        cid = lax.axis_index("c")
        sid = lax.axis_index("s")
        wid = cid * NS + sid

        zero = jnp.zeros((16,), jnp.float32)
        ones = jnp.ones((16,), jnp.float32)

        @pl.loop(0, BATCH)
        def _(r):
            @pl.loop(0, H // 16)
            def _(c):
                oh_v[r, pl.ds(c * 16, 16)] = zero

        pltpu.sync_copy(oh_v.at[pl.ds(0, CRPT)], cac.at[pl.ds(sid * CRPT, CRPT)])
        plsc.subcore_barrier()

        @pl.loop(0, NB // IDX_CHUNK)
        def _(c):
            pltpu.sync_copy(c_hbm.at[wid, pl.ds(c * IDX_CHUNK, IDX_CHUNK)], c_v)
            pltpu.sync_copy(s_hbm.at[wid, pl.ds(c * IDX_CHUNK, IDX_CHUNK)], s_v)

            @pl.loop(0, IDX_CHUNK)
            def _(j):
                for l in range(BATCH // 16):
                    rows = lax.iota(jnp.int32, 16) + (l * 16)
                    colv = c_v.at[j][pl.ds(l * 16, 16)]
                    plsc.addupdate_scatter(oh_v, [rows, colv], ones)
                pltpu.sync_copy(oh_v, cac.at[s_v.at[j]], add=True)
                for l in range(BATCH // 16):
                    rows = lax.iota(jnp.int32, 16) + (l * 16)
                    colv = c_v.at[j][pl.ds(l * 16, 16)]
                    plsc.addupdate_scatter(oh_v, [rows, colv], -ones)

        plsc.subcore_barrier()

        pltpu.sync_copy(cac.at[pl.ds(sid * CRPT, CRPT)],
                        cnt_hbm.at[cid, pl.ds(sid * CRPT, CRPT)])

    return k(c3, s3)


@jax.jit
def kernel(state_embedding, typed_edges, W):
    et = typed_edges[0] - 1
    src = typed_edges[1]
    dst = typed_edges[2]

    def shard(x, padval, nb, batch):
        pad = NW * nb * batch - E
        x = jnp.concatenate([x, jnp.full((pad,), padval, jnp.int32)])
        return x.reshape(nb, NW, batch).swapaxes(0, 1)

    g3 = shard(et * N + src, 0, RNB, RB)
    d3 = shard(dst, DUMP, RNB, RB)
    c3 = shard(((dst & 7) << 4) + et, 0, NB, BATCH)
    s3 = shard(dst >> 3, DUMP >> 3, NB, BATCH)

    wstk = jnp.concatenate([W[:, :, :H], W[:, :, H:]], axis=0)

    yy = _yy_matmul(state_embedding, wstk)
    cparts = _sc_counts(c3, s3)
    partials = _sc_rows(yy, g3, d3)
    ydst = yy.reshape(8, N, H)[T:]
    cnt = cparts.reshape(2, CROWS * 8, CNT_W)
    return _combine(partials, cnt, ydst)
